# baseline (speedup 1.0000x reference)
import functools

import jax
import jax.numpy as jnp
from jax import lax
from jax.experimental import pallas as pl
from jax.experimental.pallas import tpu as pltpu
from jax.experimental.pallas import tpu_sc as plsc

VOCAB = 100000
EMB = 300
PAD_D = 16
VBLK = 16384
NBLK = 7
LOG_VBLK = 14
LOG_JB = 11
JB = VBLK // 8
SLOT_ROWS = NBLK * VBLK // 8

NUM_CORES = 2
NUM_SUBCORES = 16
NW = NUM_CORES * NUM_SUBCORES
LANES = 16


def _mlp_table_body(tab_ref, w1_ref, b1_ref, w2_ref, b2_ref, out_ref):
    emb_t = tab_ref[...]
    h_t = jnp.dot(w1_ref[...], emb_t, preferred_element_type=jnp.float32)
    h_t = jnp.maximum(h_t + b1_ref[...], 0.0)
    o16_t = (
        jnp.dot(w2_ref[...], h_t, preferred_element_type=jnp.float32)
        + b2_ref[...]
    )
    c = VBLK // 8
    out_ref[...] = jnp.concatenate(
        [o16_t[:, j * c:(j + 1) * c].T for j in range(8)], axis=1
    )


def _fused_table(table_t, W1, b1, W2r, b2r):
    return pl.pallas_call(
        _mlp_table_body,
        grid=(NBLK,),
        in_specs=[
            pl.BlockSpec((EMB, VBLK), lambda i: (0, i)),
            pl.BlockSpec((8, EMB), lambda i: (0, 0)),
            pl.BlockSpec((8, 1), lambda i: (0, 0)),
            pl.BlockSpec((PAD_D, 8), lambda i: (0, 0)),
            pl.BlockSpec((PAD_D, 1), lambda i: (0, 0)),
        ],
        out_specs=pl.BlockSpec((VBLK // 8, 128), lambda i: (i, 0)),
        out_shape=jax.ShapeDtypeStruct((SLOT_ROWS, 128), jnp.float32),
    )(table_t, W1, b1, W2r, b2r)


def _make_gather(B, S):
    n_idx = B * S
    b_per_w = n_idx // NW
    bw = B // NW
    mesh = plsc.VectorSubcoreMesh(core_axis_name="c", subcore_axis_name="s")

    @functools.partial(
        pl.kernel,
        mesh=mesh,
        compiler_params=pltpu.CompilerParams(use_tc_tiling_on_sc=False),
        out_type=jax.ShapeDtypeStruct((S, 2 * B), jnp.float32),
        scratch_types=[
            pltpu.VMEM((b_per_w,), jnp.int32),
            pltpu.VMEM((b_per_w, PAD_D), jnp.float32),
            pltpu.VMEM((S, 2 * bw), jnp.float32),
            pltpu.SemaphoreType.DMA,
        ],
    )
    def gather(tab_hbm, idx_hbm, out_hbm, idx_v, rows_v, cmp_v, sem):
        wid = lax.axis_index("s") * NUM_CORES + lax.axis_index("c")
        base = wid * b_per_w
        pltpu.sync_copy(idx_hbm.at[pl.ds(base, b_per_w)], idx_v)

        def slotify(i, acc):
            v = idx_v[pl.ds(LANES * i, LANES)]
            g = (
                lax.shift_left(lax.shift_right_logical(v, LOG_VBLK), LOG_VBLK)
                + lax.shift_left(lax.bitwise_and(v, JB - 1), 3)
                + lax.bitwise_and(lax.shift_right_logical(v, LOG_JB), 7)
            )
            idx_v[pl.ds(LANES * i, LANES)] = g
            return acc

        lax.fori_loop(0, b_per_w // LANES, slotify, 0)

        pltpu.async_copy(tab_hbm.at[idx_v], rows_v, sem).wait()

        lane = lax.iota(jnp.int32, LANES)

        def compact(m, acc_):
            s = lax.shift_right_logical(m, 3)
            h = lax.bitwise_and(m, 7)
            rbase = (LANES * S) * h + s
            acc0 = jnp.zeros((LANES,), jnp.float32)
            acc1 = jnp.zeros((LANES,), jnp.float32)
            for l in range(LANES):
                row = rows_v[rbase + S * l, :]
                rrow = lax.rev(row, (0,))
                pick = lane == l
                if l % 2 == 0:
                    acc0 = jnp.where(pick, row, acc0)
                    acc1 = jnp.where(pick, rrow, acc1)
                else:
                    acc0 = jnp.where(pick, rrow, acc0)
                    acc1 = jnp.where(pick, row, acc1)
            cmp_v[s, pl.ds(LANES * h, LANES)] = acc0
            cmp_v[s, pl.ds(bw + LANES * h, LANES)] = acc1
            return acc_

        lax.fori_loop(0, S * (bw // LANES), compact, 0)
        pltpu.sync_copy(cmp_v, out_hbm.at[:, pl.ds(2 * bw * wid, 2 * bw)])

    return gather


def kernel(x, table, W1, b1, W2, b2):
    B, S = x.shape
    W1p = jnp.zeros((8, EMB), jnp.float32).at[:3, :].set(W1.T)
    b1p = jnp.zeros((8, 1), jnp.float32).at[:3, 0].set(b1)
    W2r = jnp.tile(jnp.pad(W2.T, ((0, 0), (0, 5))), (8, 1))
    b2r = jnp.tile(b2, 8).reshape(PAD_D, 1)

    fused = _fused_table(table.T, W1p, b1p, W2r, b2r)
    fused_lin = fused.reshape(SLOT_ROWS * 8, PAD_D)

    idx = x.reshape(-1).astype(jnp.int32)
    flat = _make_gather(B, S)(fused_lin, idx)
    bw = B // NW
    out = flat.reshape(S, NW, 2, bw).transpose(1, 3, 0, 2).reshape(B, S, 2)
    return out


# speedup vs baseline: 1.0188x; 1.0188x over previous
import functools

import jax
import jax.numpy as jnp
from jax import lax
from jax.experimental import pallas as pl
from jax.experimental.pallas import tpu as pltpu
from jax.experimental.pallas import tpu_sc as plsc

VOCAB = 100000
EMB = 300
PAD_D = 16
VBLK = 8192
NBLK = 13
LOG_VBLK = 13
LOG_JB = 10
JB = VBLK // 8
SLOT_ROWS = NBLK * VBLK // 8

NUM_CORES = 2
NUM_SUBCORES = 16
NW = NUM_CORES * NUM_SUBCORES
LANES = 16


def _mlp_table_body(tab_ref, w1_ref, b1_ref, w2_ref, b2_ref, out_ref):
    emb_t = tab_ref[...]
    h_t = jnp.dot(w1_ref[...], emb_t, preferred_element_type=jnp.float32)
    h_t = jnp.maximum(h_t + b1_ref[...], 0.0)
    o16_t = (
        jnp.dot(w2_ref[...], h_t, preferred_element_type=jnp.float32)
        + b2_ref[...]
    )
    c = VBLK // 8
    out_ref[...] = jnp.concatenate(
        [o16_t[:, j * c:(j + 1) * c].T for j in range(8)], axis=1
    )


def _fused_table(table_t, W1, b1, W2r, b2r):
    return pl.pallas_call(
        _mlp_table_body,
        grid=(NBLK,),
        in_specs=[
            pl.BlockSpec((EMB, VBLK), lambda i: (0, i)),
            pl.BlockSpec((8, EMB), lambda i: (0, 0)),
            pl.BlockSpec((8, 1), lambda i: (0, 0)),
            pl.BlockSpec((PAD_D, 8), lambda i: (0, 0)),
            pl.BlockSpec((PAD_D, 1), lambda i: (0, 0)),
        ],
        out_specs=pl.BlockSpec((VBLK // 8, 128), lambda i: (i, 0)),
        out_shape=jax.ShapeDtypeStruct((SLOT_ROWS, 128), jnp.float32),
    )(table_t, W1, b1, W2r, b2r)


def _make_gather(B, S):
    n_idx = B * S
    b_per_w = n_idx // NW
    bw = B // NW
    mesh = plsc.VectorSubcoreMesh(core_axis_name="c", subcore_axis_name="s")

    @functools.partial(
        pl.kernel,
        mesh=mesh,
        compiler_params=pltpu.CompilerParams(use_tc_tiling_on_sc=False),
        out_type=jax.ShapeDtypeStruct((S, 2 * B), jnp.float32),
        scratch_types=[
            pltpu.VMEM((b_per_w,), jnp.int32),
            pltpu.VMEM((b_per_w, PAD_D), jnp.float32),
            pltpu.VMEM((S, 2 * bw), jnp.float32),
            pltpu.SemaphoreType.DMA,
        ],
    )
    def gather(tab_hbm, idx_hbm, out_hbm, idx_v, rows_v, cmp_v, sem):
        wid = lax.axis_index("s") * NUM_CORES + lax.axis_index("c")
        base = wid * b_per_w
        pltpu.sync_copy(idx_hbm.at[pl.ds(base, b_per_w)], idx_v)

        def slotify(i, acc):
            v = idx_v[pl.ds(LANES * i, LANES)]
            g = (
                lax.shift_left(lax.shift_right_logical(v, LOG_VBLK), LOG_VBLK)
                + lax.shift_left(lax.bitwise_and(v, JB - 1), 3)
                + lax.bitwise_and(lax.shift_right_logical(v, LOG_JB), 7)
            )
            idx_v[pl.ds(LANES * i, LANES)] = g
            return acc

        lax.fori_loop(0, b_per_w // LANES, slotify, 0)

        pltpu.async_copy(tab_hbm.at[idx_v], rows_v, sem).wait()

        lane = lax.iota(jnp.int32, LANES)

        def compact(m, acc_):
            s = lax.shift_right_logical(m, 3)
            h = lax.bitwise_and(m, 7)
            rbase = (LANES * S) * h + s
            acc0 = jnp.zeros((LANES,), jnp.float32)
            acc1 = jnp.zeros((LANES,), jnp.float32)
            for l in range(LANES):
                row = rows_v[rbase + S * l, :]
                rrow = lax.rev(row, (0,))
                pick = lane == l
                if l % 2 == 0:
                    acc0 = jnp.where(pick, row, acc0)
                    acc1 = jnp.where(pick, rrow, acc1)
                else:
                    acc0 = jnp.where(pick, rrow, acc0)
                    acc1 = jnp.where(pick, row, acc1)
            cmp_v[s, pl.ds(LANES * h, LANES)] = acc0
            cmp_v[s, pl.ds(bw + LANES * h, LANES)] = acc1
            return acc_

        lax.fori_loop(0, S * (bw // LANES), compact, 0)
        pltpu.sync_copy(cmp_v, out_hbm.at[:, pl.ds(2 * bw * wid, 2 * bw)])

    return gather


def kernel(x, table, W1, b1, W2, b2):
    B, S = x.shape
    W1p = jnp.zeros((8, EMB), jnp.float32).at[:3, :].set(W1.T)
    b1p = jnp.zeros((8, 1), jnp.float32).at[:3, 0].set(b1)
    W2r = jnp.tile(jnp.pad(W2.T, ((0, 0), (0, 5))), (8, 1))
    b2r = jnp.tile(b2, 8).reshape(PAD_D, 1)

    fused = _fused_table(table.T, W1p, b1p, W2r, b2r)
    fused_lin = fused.reshape(SLOT_ROWS * 8, PAD_D)

    idx = x.reshape(-1).astype(jnp.int32)
    flat = _make_gather(B, S)(fused_lin, idx)
    bw = B // NW
    out = flat.reshape(S, NW, 2, bw).transpose(1, 3, 0, 2).reshape(B, S, 2)
    return out


# speedup vs baseline: 1.0380x; 1.0189x over previous
import functools

import jax
import jax.numpy as jnp
from jax import lax
from jax.experimental import pallas as pl
from jax.experimental.pallas import tpu as pltpu
from jax.experimental.pallas import tpu_sc as plsc

VOCAB = 100000
EMB = 300
PAD_D = 16
VBLK = 8192
NBLK = 13
LOG_VBLK = 13
LOG_JB = 10
JB = VBLK // 8
SLOT_ROWS = NBLK * VBLK // 8

NUM_CORES = 2
NUM_SUBCORES = 16
NW = NUM_CORES * NUM_SUBCORES
LANES = 16


def _mlp_table_body(tab_ref, w1_ref, b1_ref, w2_ref, b2_ref, out_ref):
    emb_t = tab_ref[...]
    h_t = jnp.dot(w1_ref[...], emb_t, preferred_element_type=jnp.float32)
    h_t = jnp.maximum(h_t + b1_ref[...], 0.0)
    o16_t = (
        jnp.dot(w2_ref[...], h_t, preferred_element_type=jnp.float32)
        + b2_ref[...]
    )
    c = VBLK // 8
    out_ref[...] = jnp.concatenate(
        [o16_t[:, j * c:(j + 1) * c].T for j in range(8)], axis=1
    )


def _fused_table(table_t, W1, b1, W2r, b2r):
    return pl.pallas_call(
        _mlp_table_body,
        grid=(NBLK,),
        in_specs=[
            pl.BlockSpec((EMB, VBLK), lambda i: (0, i)),
            pl.BlockSpec((8, EMB), lambda i: (0, 0)),
            pl.BlockSpec((8, 1), lambda i: (0, 0)),
            pl.BlockSpec((PAD_D, 8), lambda i: (0, 0)),
            pl.BlockSpec((PAD_D, 1), lambda i: (0, 0)),
        ],
        out_specs=pl.BlockSpec((VBLK // 8, 128), lambda i: (i, 0)),
        out_shape=jax.ShapeDtypeStruct((SLOT_ROWS, 128), jnp.float32),
    )(table_t, W1, b1, W2r, b2r)


def _make_gather(B, S):
    n_idx = B * S
    b_per_w = n_idx // NW
    bw = B // NW
    mesh = plsc.VectorSubcoreMesh(core_axis_name="c", subcore_axis_name="s")

    @functools.partial(
        pl.kernel,
        mesh=mesh,
        compiler_params=pltpu.CompilerParams(use_tc_tiling_on_sc=False),
        out_type=jax.ShapeDtypeStruct((S, 2 * B), jnp.float32),
        scratch_types=[
            pltpu.VMEM((b_per_w,), jnp.int32),
            pltpu.VMEM((b_per_w, PAD_D), jnp.float32),
            pltpu.VMEM((S, 2 * bw), jnp.float32),
            pltpu.SemaphoreType.DMA,
        ],
    )
    def gather(tab_hbm, idx_hbm, out_hbm, idx_v, rows_v, cmp_v, sem):
        wid = lax.axis_index("s") * NUM_CORES + lax.axis_index("c")
        base = wid * b_per_w
        pltpu.sync_copy(idx_hbm.at[pl.ds(base, b_per_w)], idx_v)

        def slotify(i, acc):
            v = idx_v[pl.ds(LANES * i, LANES)]
            g = (
                lax.shift_left(lax.shift_right_logical(v, LOG_VBLK), LOG_VBLK)
                + lax.shift_left(lax.bitwise_and(v, JB - 1), 3)
                + lax.bitwise_and(lax.shift_right_logical(v, LOG_JB), 7)
            )
            idx_v[pl.ds(LANES * i, LANES)] = g
            return acc

        lax.fori_loop(0, b_per_w // LANES, slotify, 0, unroll=4)

        pltpu.async_copy(tab_hbm.at[idx_v], rows_v, sem).wait()

        lane = lax.iota(jnp.int32, LANES)

        def compact(m, acc_):
            s = lax.shift_right_logical(m, 3)
            h = lax.bitwise_and(m, 7)
            rbase = (LANES * S) * h + s
            acc0 = jnp.zeros((LANES,), jnp.float32)
            acc1 = jnp.zeros((LANES,), jnp.float32)
            for l in range(LANES):
                row = rows_v[rbase + S * l, :]
                rrow = lax.rev(row, (0,))
                pick = lane == l
                if l % 2 == 0:
                    acc0 = jnp.where(pick, row, acc0)
                    acc1 = jnp.where(pick, rrow, acc1)
                else:
                    acc0 = jnp.where(pick, rrow, acc0)
                    acc1 = jnp.where(pick, row, acc1)
            cmp_v[s, pl.ds(LANES * h, LANES)] = acc0
            cmp_v[s, pl.ds(bw + LANES * h, LANES)] = acc1
            return acc_

        lax.fori_loop(0, S * (bw // LANES), compact, 0, unroll=2)
        pltpu.sync_copy(cmp_v, out_hbm.at[:, pl.ds(2 * bw * wid, 2 * bw)])

    return gather


def kernel(x, table, W1, b1, W2, b2):
    B, S = x.shape
    W1p = jnp.zeros((8, EMB), jnp.float32).at[:3, :].set(W1.T)
    b1p = jnp.zeros((8, 1), jnp.float32).at[:3, 0].set(b1)
    W2r = jnp.tile(jnp.pad(W2.T, ((0, 0), (0, 5))), (8, 1))
    b2r = jnp.tile(b2, 8).reshape(PAD_D, 1)

    fused = _fused_table(table.T, W1p, b1p, W2r, b2r)
    fused_lin = fused.reshape(SLOT_ROWS * 8, PAD_D)

    idx = x.reshape(-1).astype(jnp.int32)
    flat = _make_gather(B, S)(fused_lin, idx)
    bw = B // NW
    out = flat.reshape(S, NW, 2, bw).transpose(1, 3, 0, 2).reshape(B, S, 2)
    return out


# speedup vs baseline: 1.0688x; 1.0296x over previous
import functools

import jax
import jax.numpy as jnp
from jax import lax
from jax.experimental import pallas as pl
from jax.experimental.pallas import tpu as pltpu
from jax.experimental.pallas import tpu_sc as plsc

VOCAB = 100000
EMB = 300
PAD_D = 16
VBLK = 8192
NBLK = 13
LOG_VBLK = 13
LOG_JB = 10
JB = VBLK // 8
SLOT_ROWS = NBLK * VBLK // 8

NUM_CORES = 2
NUM_SUBCORES = 16
NW = NUM_CORES * NUM_SUBCORES
LANES = 16


def _mlp_table_body(tab_ref, w1_ref, b1_ref, w2_ref, b2_ref, out_ref):
    emb_t = tab_ref[...]
    w1p = jnp.concatenate(
        [w1_ref[...], jnp.zeros((5, EMB), jnp.float32)], axis=0
    )
    w2p = jnp.concatenate(
        [w2_ref[...], jnp.zeros((2, 5), jnp.float32)], axis=1
    )
    w2r = jnp.concatenate([w2p] * 8, axis=0)
    h_t = jnp.dot(w1p, emb_t, preferred_element_type=jnp.float32)
    h_t = jnp.maximum(h_t + b1_ref[...], 0.0)
    o16_t = (
        jnp.dot(w2r, h_t, preferred_element_type=jnp.float32)
        + b2_ref[...]
    )
    c = VBLK // 8
    out_ref[...] = jnp.concatenate(
        [o16_t[:, j * c:(j + 1) * c].T for j in range(8)], axis=1
    )


def _fused_table(table_t, W1, b1, W2r, b2r):
    return pl.pallas_call(
        _mlp_table_body,
        grid=(NBLK,),
        in_specs=[
            pl.BlockSpec((EMB, VBLK), lambda i: (0, i)),
            pl.BlockSpec((3, EMB), lambda i: (0, 0)),
            pl.BlockSpec((8, 1), lambda i: (0, 0)),
            pl.BlockSpec((2, 3), lambda i: (0, 0)),
            pl.BlockSpec((PAD_D, 1), lambda i: (0, 0)),
        ],
        out_specs=pl.BlockSpec((VBLK // 8, 128), lambda i: (i, 0)),
        out_shape=jax.ShapeDtypeStruct((SLOT_ROWS, 128), jnp.float32),
    )(table_t, W1, b1, W2r, b2r)


def _make_gather(B, S):
    n_idx = B * S
    b_per_w = n_idx // NW
    bw = B // NW
    mesh = plsc.VectorSubcoreMesh(core_axis_name="c", subcore_axis_name="s")

    @functools.partial(
        pl.kernel,
        mesh=mesh,
        compiler_params=pltpu.CompilerParams(use_tc_tiling_on_sc=False),
        out_type=jax.ShapeDtypeStruct((S, 2 * B), jnp.float32),
        scratch_types=[
            pltpu.VMEM((b_per_w,), jnp.int32),
            pltpu.VMEM((b_per_w, PAD_D), jnp.float32),
            pltpu.VMEM((S, 2 * bw), jnp.float32),
            pltpu.SemaphoreType.DMA,
        ],
    )
    def gather(tab_hbm, idx_hbm, out_hbm, idx_v, rows_v, cmp_v, sem):
        wid = lax.axis_index("s") * NUM_CORES + lax.axis_index("c")
        base = wid * b_per_w
        pltpu.sync_copy(idx_hbm.at[pl.ds(base, b_per_w)], idx_v)

        def slotify(i, acc):
            v = idx_v[pl.ds(LANES * i, LANES)]
            g = (
                lax.shift_left(lax.shift_right_logical(v, LOG_VBLK), LOG_VBLK)
                + lax.shift_left(lax.bitwise_and(v, JB - 1), 3)
                + lax.bitwise_and(lax.shift_right_logical(v, LOG_JB), 7)
            )
            idx_v[pl.ds(LANES * i, LANES)] = g
            return acc

        lax.fori_loop(0, b_per_w // LANES, slotify, 0, unroll=4)

        pltpu.async_copy(tab_hbm.at[idx_v], rows_v, sem).wait()

        lane = lax.iota(jnp.int32, LANES)

        def compact(m, acc_):
            s = lax.shift_right_logical(m, 3)
            h = lax.bitwise_and(m, 7)
            rbase = (LANES * S) * h + s
            acc0 = jnp.zeros((LANES,), jnp.float32)
            acc1 = jnp.zeros((LANES,), jnp.float32)
            for l in range(LANES):
                row = rows_v[rbase + S * l, :]
                rrow = lax.rev(row, (0,))
                pick = lane == l
                if l % 2 == 0:
                    acc0 = jnp.where(pick, row, acc0)
                    acc1 = jnp.where(pick, rrow, acc1)
                else:
                    acc0 = jnp.where(pick, rrow, acc0)
                    acc1 = jnp.where(pick, row, acc1)
            cmp_v[s, pl.ds(LANES * h, LANES)] = acc0
            cmp_v[s, pl.ds(bw + LANES * h, LANES)] = acc1
            return acc_

        lax.fori_loop(0, S * (bw // LANES), compact, 0, unroll=2)
        pltpu.sync_copy(cmp_v, out_hbm.at[:, pl.ds(2 * bw * wid, 2 * bw)])

    return gather


def kernel(x, table, W1, b1, W2, b2):
    B, S = x.shape
    b1p = jnp.zeros((8, 1), jnp.float32).at[:3, 0].set(b1)
    b2r = jnp.tile(b2, 8).reshape(PAD_D, 1)

    fused = _fused_table(table.T, W1.T, b1p, W2.T, b2r)
    fused_lin = fused.reshape(SLOT_ROWS * 8, PAD_D)

    idx = x.reshape(-1).astype(jnp.int32)
    flat = _make_gather(B, S)(fused_lin, idx)
    bw = B // NW
    out = flat.reshape(S, NW, 2, bw).transpose(1, 3, 0, 2).reshape(B, S, 2)
    return out


# speedup vs baseline: 1.0703x; 1.0014x over previous
import functools

import jax
import jax.numpy as jnp
from jax import lax
from jax.experimental import pallas as pl
from jax.experimental.pallas import tpu as pltpu
from jax.experimental.pallas import tpu_sc as plsc

VOCAB = 100000
EMB = 300
PAD_D = 16
VBLK = 8192
NBLK = 13
LOG_VBLK = 13
LOG_JB = 10
JB = VBLK // 8
SLOT_ROWS = NBLK * VBLK // 8

NUM_CORES = 2
NUM_SUBCORES = 16
NW = NUM_CORES * NUM_SUBCORES
LANES = 16


def _mlp_table_body(tab_ref, w1_ref, b1_ref, w2_ref, b2_ref, out_ref):
    emb_t = tab_ref[...]
    w1p = jnp.concatenate(
        [w1_ref[...], jnp.zeros((5, EMB), jnp.float32)], axis=0
    )
    w2p = jnp.concatenate(
        [w2_ref[...], jnp.zeros((2, 5), jnp.float32)], axis=1
    )
    w2r = jnp.concatenate([w2p] * 8, axis=0)
    h_t = jnp.dot(w1p, emb_t, preferred_element_type=jnp.float32)
    h_t = jnp.maximum(h_t + b1_ref[...], 0.0)
    o16_t = (
        jnp.dot(w2r, h_t, preferred_element_type=jnp.float32)
        + b2_ref[...]
    )
    c = VBLK // 8
    out_ref[...] = jnp.concatenate(
        [o16_t[:, j * c:(j + 1) * c].T for j in range(8)], axis=1
    )


def _fused_table(table_t, W1, b1, W2r, b2r):
    return pl.pallas_call(
        _mlp_table_body,
        grid=(NBLK,),
        in_specs=[
            pl.BlockSpec((EMB, VBLK), lambda i: (0, i)),
            pl.BlockSpec((3, EMB), lambda i: (0, 0)),
            pl.BlockSpec((8, 1), lambda i: (0, 0)),
            pl.BlockSpec((2, 3), lambda i: (0, 0)),
            pl.BlockSpec((PAD_D, 1), lambda i: (0, 0)),
        ],
        out_specs=pl.BlockSpec((VBLK // 8, 128), lambda i: (i, 0)),
        out_shape=jax.ShapeDtypeStruct((SLOT_ROWS, 128), jnp.float32),
    )(table_t, W1, b1, W2r, b2r)


def _make_gather(B, S):
    n_idx = B * S
    b_per_w = n_idx // NW
    bw = B // NW
    mesh = plsc.VectorSubcoreMesh(core_axis_name="c", subcore_axis_name="s")

    @functools.partial(
        pl.kernel,
        mesh=mesh,
        compiler_params=pltpu.CompilerParams(use_tc_tiling_on_sc=False),
        out_type=jax.ShapeDtypeStruct((S, 2 * B), jnp.float32),
        scratch_types=[
            pltpu.VMEM((b_per_w,), jnp.int32),
            pltpu.VMEM((b_per_w, PAD_D), jnp.float32),
            pltpu.VMEM((S, 2 * bw), jnp.float32),
            pltpu.SemaphoreType.DMA,
            pltpu.SemaphoreType.DMA,
        ],
    )
    def gather(tab_hbm, idx_hbm, out_hbm, idx_v, rows_v, cmp_v, sem, sem2):
        wid = lax.axis_index("s") * NUM_CORES + lax.axis_index("c")
        base = wid * b_per_w
        pltpu.sync_copy(idx_hbm.at[pl.ds(base, b_per_w)], idx_v)

        def slotify(i, acc):
            v = idx_v[pl.ds(LANES * i, LANES)]
            g = (
                lax.shift_left(lax.shift_right_logical(v, LOG_VBLK), LOG_VBLK)
                + lax.shift_left(lax.bitwise_and(v, JB - 1), 3)
                + lax.bitwise_and(lax.shift_right_logical(v, LOG_JB), 7)
            )
            idx_v[pl.ds(LANES * i, LANES)] = g
            return acc

        lax.fori_loop(0, b_per_w // LANES, slotify, 0, unroll=4)

        half = b_per_w // 2
        cp1 = pltpu.async_copy(
            tab_hbm.at[idx_v.at[pl.ds(0, half)]], rows_v.at[pl.ds(0, half)], sem
        )
        cp2 = pltpu.async_copy(
            tab_hbm.at[idx_v.at[pl.ds(half, half)]],
            rows_v.at[pl.ds(half, half)],
            sem2,
        )

        lane = lax.iota(jnp.int32, LANES)

        def compact(m, acc_):
            s = lax.shift_right_logical(m, 2)
            h = lax.bitwise_and(m, 3) + acc_
            rbase = (LANES * S) * h + s
            acc0 = jnp.zeros((LANES,), jnp.float32)
            acc1 = jnp.zeros((LANES,), jnp.float32)
            for l in range(LANES):
                row = rows_v[rbase + S * l, :]
                rrow = lax.rev(row, (0,))
                pick = lane == l
                if l % 2 == 0:
                    acc0 = jnp.where(pick, row, acc0)
                    acc1 = jnp.where(pick, rrow, acc1)
                else:
                    acc0 = jnp.where(pick, rrow, acc0)
                    acc1 = jnp.where(pick, row, acc1)
            cmp_v[s, pl.ds(LANES * h, LANES)] = acc0
            cmp_v[s, pl.ds(bw + LANES * h, LANES)] = acc1
            return acc_

        nhalf = S * (bw // LANES) // 2
        cp1.wait()
        lax.fori_loop(0, nhalf, compact, 0, unroll=2)
        cp2.wait()
        lax.fori_loop(0, nhalf, compact, 4, unroll=2)
        pltpu.sync_copy(cmp_v, out_hbm.at[:, pl.ds(2 * bw * wid, 2 * bw)])

    return gather


def kernel(x, table, W1, b1, W2, b2):
    B, S = x.shape
    b1p = jnp.zeros((8, 1), jnp.float32).at[:3, 0].set(b1)
    b2r = jnp.tile(b2, 8).reshape(PAD_D, 1)

    fused = _fused_table(table.T, W1.T, b1p, W2.T, b2r)
    fused_lin = fused.reshape(SLOT_ROWS * 8, PAD_D)

    idx = x.reshape(-1).astype(jnp.int32)
    flat = _make_gather(B, S)(fused_lin, idx)
    bw = B // NW
    out = flat.reshape(S, NW, 2, bw).transpose(1, 3, 0, 2).reshape(B, S, 2)
    return out
